# Initial kernel scaffold; baseline (speedup 1.0000x reference)
import jax
import jax.numpy as jnp
from jax.experimental import pallas as pl


def kernel(E, M, A):
    raise NotImplementedError("write your pallas kernel here")


# speedup vs baseline: 1.8734x; 1.8734x over previous
import functools

import jax
import jax.numpy as jnp
from jax.experimental import pallas as pl
from jax.experimental.pallas import tpu as pltpu


def _ceil_to(x, m):
    return ((x + m - 1) // m) * m


def _loss_kernel(eot_ref, e_ref, m_ref, a_ref, out_ref, *, d_p):
    a = a_ref[0].astype(jnp.bfloat16)

    c = jnp.dot(eot_ref[...], a, preferred_element_type=jnp.float32)
    sum_a = jnp.sum(c[d_p:d_p + 1, :])

    b = jnp.dot(c.astype(jnp.bfloat16), e_ref[...],
                preferred_element_type=jnp.float32)

    mk = m_ref[0]
    g = jnp.dot(eot_ref[0:d_p, :], e_ref[...],
                preferred_element_type=jnp.float32)
    y1 = jnp.dot(g, mk, preferred_element_type=jnp.float32)
    y2 = jnp.dot(mk, g, preferred_element_type=jnp.float32)
    t3 = jnp.sum(y1 * y2)

    dot_bm = jnp.sum(b[0:d_p, :] * mk)
    val = sum_a - 2.0 * dot_bm + t3
    out_ref[...] = val + jnp.zeros((1, 1, 128), jnp.float32)


def kernel(E, M, A):
    n, d = E.shape
    K = M.shape[0]

    E = E.astype(jnp.float32)
    norms = jnp.sqrt(jnp.sum(E * E, axis=1, keepdims=True))
    E_n = E / jnp.maximum(norms, 1e-12)

    n_p = _ceil_to(n, 128)
    d_p = _ceil_to(d, 128)
    rows = d_p + 8

    eot = jnp.zeros((rows, n_p), jnp.float32)
    eot = eot.at[:d, :n].set(E_n.T)
    eot = eot.at[d_p, :n].set(1.0)
    eot_bf = eot.astype(jnp.bfloat16)

    e_pad = jnp.zeros((n_p, d_p), jnp.float32).at[:n, :d].set(E_n)
    e_bf = e_pad.astype(jnp.bfloat16)

    M_p = M.astype(jnp.float32)
    A_p = A
    if d_p != d:
        M_p = jnp.pad(M_p, ((0, 0), (0, d_p - d), (0, d_p - d)))
    if n_p != n:
        A_p = jnp.pad(A_p, ((0, 0), (0, n_p - n), (0, n_p - n)))

    out = pl.pallas_call(
        functools.partial(_loss_kernel, d_p=d_p),
        out_shape=jax.ShapeDtypeStruct((K, 1, 128), jnp.float32),
        grid=(K,),
        in_specs=[
            pl.BlockSpec((rows, n_p), lambda k: (0, 0)),
            pl.BlockSpec((n_p, d_p), lambda k: (0, 0)),
            pl.BlockSpec((1, d_p, d_p), lambda k: (k, 0, 0)),
            pl.BlockSpec((1, n_p, n_p), lambda k: (k, 0, 0)),
        ],
        out_specs=pl.BlockSpec((1, 1, 128), lambda k: (k, 0, 0)),
        compiler_params=pltpu.CompilerParams(
            dimension_semantics=("parallel",),
            vmem_limit_bytes=48 * 2 ** 20,
        ),
    )(eot_bf, e_bf, M_p, A_p)

    return jnp.sum(out[:, 0, 0])
